# baseline (speedup 1.0000x reference)
import functools

import jax
import jax.numpy as jnp
from jax import lax
from jax.experimental import pallas as pl
from jax.experimental.pallas import tpu as pltpu
from jax.experimental.pallas import tpu_sc as plsc

_K = 24
_SET = 1
_NW = 32


def _make_gather(N_IN, N_OUT, B, C):
    rows_w = N_OUT // _NW
    rem = N_OUT - rows_w * _NW
    n_full = rows_w // _K
    ctail = rows_w - n_full * _K
    slab = (rows_w + rem + 7) // 8 * 8
    n_groups = (n_full + _SET - 1) // _SET
    n_super = (n_groups + 3 + 1) // 2
    mesh = plsc.VectorSubcoreMesh(core_axis_name="c", subcore_axis_name="s")

    @functools.partial(
        pl.kernel,
        mesh=mesh,
        out_type=jax.ShapeDtypeStruct((N_OUT, B, C), jnp.float32),
        scratch_types=[
            pltpu.VMEM((slab,), jnp.int32),
            pltpu.VMEM((2 * _SET, _K, B, C), jnp.float32),
        ]
        + [pltpu.SemaphoreType.DMA] * (4 * _SET + 1),
    )
    def k(x_hbm, idx_hbm, out_hbm, idx_v, rows_v, *sems):
        gsem = sems[: 2 * _SET]
        ssem = sems[2 * _SET : 4 * _SET]
        xsem = sems[4 * _SET]
        wid = lax.axis_index("s") * 2 + lax.axis_index("c")
        row0 = wid * rows_w

        pltpu.sync_copy(idx_hbm.at[pl.ds(row0, slab)], idx_v)

        def run_group(g, sb):
            osb = _SET - sb
            for i in range(_SET):
                s = sb + i
                f = (g - 2) * _SET + i

                @pl.when(jnp.logical_and(f >= 0, f < n_full))
                def _():
                    pltpu.make_async_copy(
                        rows_v.at[s], out_hbm.at[pl.ds(0, _K)], ssem[s]
                    ).wait()

                l = g * _SET + i

                @pl.when(l < n_full)
                def _():
                    pltpu.async_copy(
                        x_hbm.at[idx_v.at[pl.ds(l * _K, _K)]],
                        rows_v.at[s],
                        gsem[s],
                    )

            for i in range(_SET):
                s = osb + i
                p = (g - 1) * _SET + i

                @pl.when(jnp.logical_and(p >= 0, p < n_full))
                def _():
                    pltpu.make_async_copy(
                        x_hbm.at[idx_v.at[pl.ds(0, _K)]],
                        rows_v.at[s],
                        gsem[s],
                    ).wait()
                    pltpu.async_copy(
                        rows_v.at[s],
                        out_hbm.at[pl.ds(row0 + p * _K, _K)],
                        ssem[s],
                    )

        def super_group(h, carry):
            run_group(2 * h, 0)
            run_group(2 * h + 1, _SET)
            return carry

        lax.fori_loop(0, n_super, super_group, 0)

        if ctail:

            def _():
                pltpu.async_copy(
                    x_hbm.at[idx_v.at[pl.ds(n_full * _K, ctail)]],
                    rows_v.at[0].at[pl.ds(0, ctail)],
                    xsem,
                ).wait()
                pltpu.async_copy(
                    rows_v.at[0].at[pl.ds(0, ctail)],
                    out_hbm.at[pl.ds(row0 + n_full * _K, ctail)],
                    xsem,
                ).wait()

            _()

        if rem:

            @pl.when(wid == _NW - 1)
            def _():
                pltpu.async_copy(
                    x_hbm.at[idx_v.at[pl.ds(rows_w, rem)]],
                    rows_v.at[0].at[pl.ds(0, rem)],
                    xsem,
                ).wait()
                pltpu.async_copy(
                    rows_v.at[0].at[pl.ds(0, rem)],
                    out_hbm.at[pl.ds(row0 + rows_w, rem)],
                    xsem,
                ).wait()

    return k


def kernel(x, center_idx):
    B, N_IN, C = x.shape
    N_OUT = center_idx.shape[0]
    rows_w = N_OUT // _NW
    rem = N_OUT - rows_w * _NW
    slab = (rows_w + rem + 7) // 8 * 8
    idx_len = (_NW - 1) * rows_w + slab

    idx = center_idx.astype(jnp.int32)
    if idx_len > N_OUT:
        idx = jnp.concatenate([idx, jnp.zeros((idx_len - N_OUT,), jnp.int32)])

    x_t = jnp.transpose(x, (1, 0, 2))
    out_t = _make_gather(N_IN, N_OUT, B, C)(x_t, idx)
    return jnp.transpose(out_t, (1, 0, 2))


# speedup vs baseline: 1.0048x; 1.0048x over previous
import functools

import jax
import jax.numpy as jnp
from jax import lax
from jax.experimental import pallas as pl
from jax.experimental.pallas import tpu as pltpu
from jax.experimental.pallas import tpu_sc as plsc

_K = 16
_SET = 1
_NW = 32


def _make_gather(N_IN, N_OUT, B, C):
    rows_w = N_OUT // _NW
    rem = N_OUT - rows_w * _NW
    n_full = rows_w // _K
    ctail = rows_w - n_full * _K
    slab = (rows_w + rem + 7) // 8 * 8
    n_groups = (n_full + _SET - 1) // _SET
    n_super = (n_groups + 3 + 1) // 2
    mesh = plsc.VectorSubcoreMesh(core_axis_name="c", subcore_axis_name="s")

    @functools.partial(
        pl.kernel,
        mesh=mesh,
        out_type=jax.ShapeDtypeStruct((N_OUT, B, C), jnp.float32),
        scratch_types=[
            pltpu.VMEM((slab,), jnp.int32),
            pltpu.VMEM((2 * _SET, _K, B, C), jnp.float32),
        ]
        + [pltpu.SemaphoreType.DMA] * (4 * _SET + 1),
    )
    def k(x_hbm, idx_hbm, out_hbm, idx_v, rows_v, *sems):
        gsem = sems[: 2 * _SET]
        ssem = sems[2 * _SET : 4 * _SET]
        xsem = sems[4 * _SET]
        wid = lax.axis_index("s") * 2 + lax.axis_index("c")
        row0 = wid * rows_w

        pltpu.sync_copy(idx_hbm.at[pl.ds(row0, slab)], idx_v)

        def run_group(g, sb):
            osb = _SET - sb
            for i in range(_SET):
                s = sb + i
                f = (g - 2) * _SET + i

                @pl.when(jnp.logical_and(f >= 0, f < n_full))
                def _():
                    pltpu.make_async_copy(
                        rows_v.at[s], out_hbm.at[pl.ds(0, _K)], ssem[s]
                    ).wait()

                l = g * _SET + i

                @pl.when(l < n_full)
                def _():
                    pltpu.async_copy(
                        x_hbm.at[idx_v.at[pl.ds(l * _K, _K)]],
                        rows_v.at[s],
                        gsem[s],
                    )

            for i in range(_SET):
                s = osb + i
                p = (g - 1) * _SET + i

                @pl.when(jnp.logical_and(p >= 0, p < n_full))
                def _():
                    pltpu.make_async_copy(
                        x_hbm.at[idx_v.at[pl.ds(0, _K)]],
                        rows_v.at[s],
                        gsem[s],
                    ).wait()
                    pltpu.async_copy(
                        rows_v.at[s],
                        out_hbm.at[pl.ds(row0 + p * _K, _K)],
                        ssem[s],
                    )

        def super_group(h, carry):
            run_group(2 * h, 0)
            run_group(2 * h + 1, _SET)
            return carry

        lax.fori_loop(0, n_super, super_group, 0)

        if ctail:

            def _():
                pltpu.async_copy(
                    x_hbm.at[idx_v.at[pl.ds(n_full * _K, ctail)]],
                    rows_v.at[0].at[pl.ds(0, ctail)],
                    xsem,
                ).wait()
                pltpu.async_copy(
                    rows_v.at[0].at[pl.ds(0, ctail)],
                    out_hbm.at[pl.ds(row0 + n_full * _K, ctail)],
                    xsem,
                ).wait()

            _()

        if rem:

            @pl.when(wid == _NW - 1)
            def _():
                pltpu.async_copy(
                    x_hbm.at[idx_v.at[pl.ds(rows_w, rem)]],
                    rows_v.at[0].at[pl.ds(0, rem)],
                    xsem,
                ).wait()
                pltpu.async_copy(
                    rows_v.at[0].at[pl.ds(0, rem)],
                    out_hbm.at[pl.ds(row0 + rows_w, rem)],
                    xsem,
                ).wait()

    return k


def kernel(x, center_idx):
    B, N_IN, C = x.shape
    N_OUT = center_idx.shape[0]
    rows_w = N_OUT // _NW
    rem = N_OUT - rows_w * _NW
    slab = (rows_w + rem + 7) // 8 * 8
    idx_len = (_NW - 1) * rows_w + slab

    idx = center_idx.astype(jnp.int32)
    if idx_len > N_OUT:
        idx = jnp.concatenate([idx, jnp.zeros((idx_len - N_OUT,), jnp.int32)])

    x_t = jnp.transpose(x, (1, 0, 2))
    out_t = _make_gather(N_IN, N_OUT, B, C)(x_t, idx)
    return jnp.transpose(out_t, (1, 0, 2))
